# baseline (speedup 1.0000x reference)
import dataclasses
import functools

import jax
import jax.numpy as jnp
from jax.experimental import pallas as pl
from jax.experimental.pallas import tpu as pltpu
from jax.experimental.pallas import tpu_sc as plsc

IN_F = 4096
OUT_F = 1024
BATCH = 16384
ROWS_PER_STEP = 8
LANES = 16


def _argmax_body(w_ref, o_ref):
    w = w_ref[...]
    m = jnp.max(w, axis=1, keepdims=True)
    ii = jax.lax.broadcasted_iota(jnp.int32, w.shape, 1)
    cand = jnp.where(w == m, ii, IN_F)
    am = jnp.min(cand, axis=1).astype(jnp.int32)
    o_ref[...] = am.reshape(OUT_F // 128, 128)


def _argmax(raw_weight):
    return pl.pallas_call(
        _argmax_body,
        out_shape=jax.ShapeDtypeStruct((OUT_F // 128, 128), jnp.int32),
    )(raw_weight)


def _gather_sc(x, idx):
    mesh = plsc.VectorSubcoreMesh(core_axis_name="c", subcore_axis_name="s")
    cp = pltpu.CompilerParams()
    if "needs_layout_passes" in pltpu.CompilerParams.__dataclass_fields__:
        cp = dataclasses.replace(cp, needs_layout_passes=False)

    n_tiles = 32
    rows_per_tile = BATCH // n_tiles
    n_chunks = rows_per_tile // ROWS_PER_STEP

    @functools.partial(
        pl.kernel,
        mesh=mesh,
        out_type=jax.ShapeDtypeStruct((BATCH, OUT_F), jnp.float32),
        scratch_types=[
            pltpu.VMEM((OUT_F // 128, 128), jnp.int32),
            pltpu.VMEM((1, OUT_F), jnp.int32),
            pltpu.VMEM((2, ROWS_PER_STEP, IN_F), jnp.float32),
            pltpu.VMEM((2, ROWS_PER_STEP, OUT_F), jnp.float32),
        ] + [pltpu.SemaphoreType.DMA] * 18,
        compiler_params=cp,
    )
    def k(i_hbm, x_hbm, o_hbm, i8_vmem, i_vmem, xb, ob, *sems):
        pltpu.async_copy(i_hbm, i8_vmem, sems[0]).wait()
        for rr in range(OUT_F // 128):
            for kk in range(128 // LANES):
                i_vmem[0, pl.ds(rr * 128 + kk * LANES, LANES)] = (
                    i8_vmem[rr, pl.ds(kk * LANES, LANES)])
        idx_ref = i_vmem.at[0]
        n_split = 8
        part = ROWS_PER_STEP // n_split
        sin = (sems[0:n_split], sems[n_split:2 * n_split])
        sout = (sems[2 * n_split], sems[2 * n_split + 1])

        wid = jax.lax.axis_index("s") * 2 + jax.lax.axis_index("c")
        base = wid * rows_per_tile

        def in_copies(chunk, buf):
            row0 = base + chunk * ROWS_PER_STEP
            return tuple(
                pltpu.make_async_copy(
                    x_hbm.at[pl.ds(row0 + p * part, part)],
                    xb.at[buf].at[pl.ds(p * part, part)], sin[buf][p])
                for p in range(n_split))

        def out_copy(chunk, buf):
            return pltpu.make_async_copy(
                ob.at[buf],
                o_hbm.at[pl.ds(base + chunk * ROWS_PER_STEP, ROWS_PER_STEP)],
                sout[buf])

        for c in in_copies(0, 0):
            c.start()

        @pl.loop(0, n_chunks, step=2)
        def _(g):
            for b in range(2):
                gi = g + b

                @pl.when(gi + 1 < n_chunks)
                def _():
                    for c in in_copies(gi + 1, 1 - b):
                        c.start()

                for c in in_copies(gi, b):
                    c.wait()

                @pl.when(gi >= 2)
                def _():
                    out_copy(gi - 2, b).wait()

                x_vmem = xb.at[b]
                o_vmem = ob.at[b]

                @plsc.parallel_loop(0, OUT_F // LANES, unroll=8)
                def _(j):
                    cols = idx_ref[pl.ds(j * LANES, LANES)]
                    for r in range(ROWS_PER_STEP):
                        rows = jnp.full((LANES,), r, jnp.int32)
                        vals = plsc.load_gather(x_vmem, [rows, cols])
                        o_vmem[r, pl.ds(j * LANES, LANES)] = vals

                out_copy(gi, b).start()

        out_copy(n_chunks - 2, 0).wait()
        out_copy(n_chunks - 1, 1).wait()

    return k(idx, x)


def kernel(x, raw_weight):
    idx = _argmax(raw_weight)
    return _gather_sc(x, idx)


# speedup vs baseline: 1.0191x; 1.0191x over previous
import dataclasses
import functools

import jax
import jax.numpy as jnp
from jax.experimental import pallas as pl
from jax.experimental.pallas import tpu as pltpu
from jax.experimental.pallas import tpu_sc as plsc

IN_F = 4096
OUT_F = 1024
BATCH = 16384
ROWS_PER_STEP = 8
LANES = 16


def _argmax_body(w_ref, o_ref):
    w = w_ref[...]
    m = jnp.max(w, axis=1, keepdims=True)
    ii = jax.lax.broadcasted_iota(jnp.int32, w.shape, 1)
    cand = jnp.where(w == m, ii, IN_F)
    am = jnp.min(cand, axis=1).astype(jnp.int32)
    o_ref[...] = am.reshape(OUT_F // 128, 128)


def _argmax(raw_weight):
    return pl.pallas_call(
        _argmax_body,
        out_shape=jax.ShapeDtypeStruct((OUT_F // 128, 128), jnp.int32),
    )(raw_weight)


def _gather_sc(x, idx):
    mesh = plsc.VectorSubcoreMesh(core_axis_name="c", subcore_axis_name="s")
    cp = pltpu.CompilerParams()
    if "needs_layout_passes" in pltpu.CompilerParams.__dataclass_fields__:
        cp = dataclasses.replace(cp, needs_layout_passes=False)

    n_tiles = 32
    rows_per_tile = BATCH // n_tiles
    n_chunks = rows_per_tile // ROWS_PER_STEP

    @functools.partial(
        pl.kernel,
        mesh=mesh,
        out_type=jax.ShapeDtypeStruct((BATCH, OUT_F), jnp.float32),
        scratch_types=[
            pltpu.VMEM((OUT_F // 128, 128), jnp.int32),
            pltpu.VMEM((1, OUT_F), jnp.int32),
            pltpu.VMEM((2, ROWS_PER_STEP, IN_F), jnp.float32),
            pltpu.VMEM((2, ROWS_PER_STEP, OUT_F), jnp.float32),
        ] + [pltpu.SemaphoreType.DMA] * 12,
        compiler_params=cp,
    )
    def k(i_hbm, x_hbm, o_hbm, i8_vmem, i_vmem, xb, ob, *sems):
        pltpu.async_copy(i_hbm, i8_vmem, sems[0]).wait()
        for rr in range(OUT_F // 128):
            for kk in range(128 // LANES):
                i_vmem[0, pl.ds(rr * 128 + kk * LANES, LANES)] = (
                    i8_vmem[rr, pl.ds(kk * LANES, LANES)])
        idx_ref = i_vmem.at[0]
        n_split = 4
        part = ROWS_PER_STEP // n_split
        sin = (sems[0:n_split], sems[n_split:2 * n_split])
        sout = (sems[2 * n_split:2 * n_split + 2],
                sems[2 * n_split + 2:2 * n_split + 4])
        ohalf = ROWS_PER_STEP // 2

        wid = jax.lax.axis_index("s") * 2 + jax.lax.axis_index("c")
        base = wid * rows_per_tile

        def in_copies(chunk, buf):
            row0 = base + chunk * ROWS_PER_STEP
            return tuple(
                pltpu.make_async_copy(
                    x_hbm.at[pl.ds(row0 + p * part, part)],
                    xb.at[buf].at[pl.ds(p * part, part)], sin[buf][p])
                for p in range(n_split))

        def out_copies(chunk, buf):
            row0 = base + chunk * ROWS_PER_STEP
            return tuple(
                pltpu.make_async_copy(
                    ob.at[buf].at[pl.ds(p * ohalf, ohalf)],
                    o_hbm.at[pl.ds(row0 + p * ohalf, ohalf)], sout[buf][p])
                for p in range(2))

        for c in in_copies(0, 0):
            c.start()

        @pl.loop(0, n_chunks, step=2)
        def _(g):
            for b in range(2):
                gi = g + b

                @pl.when(gi + 1 < n_chunks)
                def _():
                    for c in in_copies(gi + 1, 1 - b):
                        c.start()

                for c in in_copies(gi, b):
                    c.wait()

                @pl.when(gi >= 2)
                def _():
                    for c in out_copies(gi - 2, b):
                        c.wait()

                x_vmem = xb.at[b]
                o_vmem = ob.at[b]

                @plsc.parallel_loop(0, OUT_F // LANES, unroll=8)
                def _(j):
                    cols = idx_ref[pl.ds(j * LANES, LANES)]
                    for r in range(ROWS_PER_STEP):
                        rows = jnp.full((LANES,), r, jnp.int32)
                        vals = plsc.load_gather(x_vmem, [rows, cols])
                        o_vmem[r, pl.ds(j * LANES, LANES)] = vals

                for c in out_copies(gi, b):
                    c.start()

        for c in out_copies(n_chunks - 2, 0):
            c.wait()
        for c in out_copies(n_chunks - 1, 1):
            c.wait()

    return k(idx, x)


def kernel(x, raw_weight):
    idx = _argmax(raw_weight)
    return _gather_sc(x, idx)
